# baseline (speedup 1.0000x reference)
import functools

import numpy as np
import jax
import jax.numpy as jnp
from jax import lax
from jax.experimental import pallas as pl
from jax.experimental.pallas import tpu as pltpu
from jax.experimental.pallas import tpu_sc as plsc

N, L, D = 64, 577, 768
KEEP = 288
OUT_L = KEEP + 1
ROWS = N * OUT_L
NW = 32
PER_W = 584
STRIDE = 578
CHUNK = 72
CHUNK_SIZES = [CHUNK] * 8 + [PER_W - 8 * CHUNK]
CHUNK_OFFS = [i * CHUNK for i in range(9)]


def _rotl32(x, d):
    return ((x << np.uint32(d)) | (x >> np.uint32(32 - d))).astype(np.uint32)


def _threefry2x32(k0, k1, x0, x1):
    rot = ((13, 15, 26, 6), (17, 29, 16, 24))
    ks = (np.uint32(k0), np.uint32(k1),
          np.uint32(np.uint32(k0) ^ np.uint32(k1) ^ np.uint32(0x1BD11BDA)))
    x0 = (x0 + ks[0]).astype(np.uint32)
    x1 = (x1 + ks[1]).astype(np.uint32)
    for i in range(5):
        for r in rot[i % 2]:
            x0 = (x0 + x1).astype(np.uint32)
            x1 = _rotl32(x1, r)
            x1 = (x0 ^ x1).astype(np.uint32)
        x0 = (x0 + ks[(i + 1) % 3]).astype(np.uint32)
        x1 = (x1 + ks[(i + 2) % 3] + np.uint32(i + 1)).astype(np.uint32)
    return x0, x1


def _uniform_key1(shape):
    n = int(np.prod(shape))
    hi = np.zeros(n, dtype=np.uint32)
    lo = np.arange(n, dtype=np.uint32)
    o0, o1 = _threefry2x32(0, 1, hi, lo)
    bits = (o0 ^ o1).astype(np.uint32)
    fb = (bits >> np.uint32(9)) | np.uint32(0x3F800000)
    return (fb.view(np.float32) - np.float32(1.0)).reshape(shape)


def _build_indices():
    scores = _uniform_key1((N, L - 1))
    pm = np.argsort(scores, axis=1, kind="stable")[:, :KEEP] + 1
    pm.sort(axis=1)
    mask = np.concatenate([np.zeros((N, 1), pm.dtype), pm], axis=1)
    flat = (np.arange(N, dtype=mask.dtype)[:, None] * L + mask).reshape(-1)
    bases = (STRIDE * np.arange(NW)) // 8 * 8
    gather_pos = bases[:, None] + np.arange(PER_W)[None, :]
    return np.ascontiguousarray(flat[gather_pos].astype(np.int32))


_IDX2D = _build_indices()


@functools.cache
def _make_sc_gather():
    @functools.partial(
        pl.kernel,
        mesh=plsc.VectorSubcoreMesh(core_axis_name="c", subcore_axis_name="s"),
        out_type=jax.ShapeDtypeStruct((ROWS, D), jnp.float32),
        scratch_types=[
            pltpu.VMEM((PER_W,), jnp.int32),
            pltpu.VMEM((CHUNK, D), jnp.float32),
            pltpu.VMEM((CHUNK, D), jnp.float32),
            pltpu.SemaphoreType.DMA,
            pltpu.SemaphoreType.DMA,
            pltpu.SemaphoreType.DMA,
            pltpu.SemaphoreType.DMA,
        ],
    )
    def _sc_gather(x_hbm, idx_hbm, out_hbm, idx_v, buf0, buf1, g0, g1, s0, s1):
        wid = lax.axis_index("s") * 2 + lax.axis_index("c")
        base = (STRIDE * wid) // 8 * 8
        pltpu.sync_copy(idx_hbm.at[wid], idx_v)

        bufs = (buf0, buf1)
        gsems = (g0, g1)
        ssems = (s0, s1)
        gather_h = [None, None]
        scatter_h = [None, None]

        def start_gather(c):
            b = c & 1
            sz = CHUNK_SIZES[c]
            idx_slice = idx_v.at[pl.ds(CHUNK_OFFS[c], sz)]
            gather_h[b] = pltpu.async_copy(
                x_hbm.at[idx_slice], bufs[b].at[pl.ds(0, sz)], gsems[b])

        start_gather(0)
        for c in range(9):
            b = c & 1
            gather_h[b].wait()
            if c + 1 < 9:
                if scatter_h[1 - b] is not None:
                    scatter_h[1 - b].wait()
                start_gather(c + 1)
            sz = CHUNK_SIZES[c]
            scatter_h[b] = pltpu.async_copy(
                bufs[b].at[pl.ds(0, sz)],
                out_hbm.at[pl.ds(base + CHUNK_OFFS[c], sz)],
                ssems[b])
        scatter_h[0].wait()
        scatter_h[1].wait()

    return _sc_gather


def kernel(x):
    x_flat = x.reshape(N * L, D)
    idx2d = jnp.asarray(_IDX2D)
    out = _make_sc_gather()(x_flat, idx2d)
    return out.reshape(N, OUT_L, D)


# speedup vs baseline: 5.7509x; 5.7509x over previous
import functools

import numpy as np
import jax
import jax.numpy as jnp
from jax import lax
from jax.experimental import pallas as pl
from jax.experimental.pallas import tpu as pltpu
from jax.experimental.pallas import tpu_sc as plsc

N, L, D = 64, 577, 768
KEEP = 288
OUT_L = KEEP + 1
ROWS = N * OUT_L
NW = 32
PER_W = 584
STRIDE = 578
CHUNK = 72
CHUNK_SIZES = [CHUNK] * 8 + [PER_W - 8 * CHUNK]
CHUNK_OFFS = [i * CHUNK for i in range(9)]


def _rotl32(x, d):
    return ((x << np.uint32(d)) | (x >> np.uint32(32 - d))).astype(np.uint32)


def _threefry2x32(k0, k1, x0, x1):
    rot = ((13, 15, 26, 6), (17, 29, 16, 24))
    ks = (np.uint32(k0), np.uint32(k1),
          np.uint32(np.uint32(k0) ^ np.uint32(k1) ^ np.uint32(0x1BD11BDA)))
    x0 = (x0 + ks[0]).astype(np.uint32)
    x1 = (x1 + ks[1]).astype(np.uint32)
    for i in range(5):
        for r in rot[i % 2]:
            x0 = (x0 + x1).astype(np.uint32)
            x1 = _rotl32(x1, r)
            x1 = (x0 ^ x1).astype(np.uint32)
        x0 = (x0 + ks[(i + 1) % 3]).astype(np.uint32)
        x1 = (x1 + ks[(i + 2) % 3] + np.uint32(i + 1)).astype(np.uint32)
    return x0, x1


def _uniform_key1(shape):
    n = int(np.prod(shape))
    hi = np.zeros(n, dtype=np.uint32)
    lo = np.arange(n, dtype=np.uint32)
    o0, o1 = _threefry2x32(0, 1, hi, lo)
    bits = (o0 ^ o1).astype(np.uint32)
    fb = (bits >> np.uint32(9)) | np.uint32(0x3F800000)
    return (fb.view(np.float32) - np.float32(1.0)).reshape(shape)


def _build_indices():
    scores = _uniform_key1((N, L - 1))
    pm = np.argsort(scores, axis=1, kind="stable")[:, :KEEP] + 1
    pm.sort(axis=1)
    mask = np.concatenate([np.zeros((N, 1), pm.dtype), pm], axis=1)
    bases = (STRIDE * np.arange(NW)) // 8 * 8
    j = bases[:, None] + np.arange(PER_W)[None, :]
    i_pos = j // N
    n_pos = j % N
    return np.ascontiguousarray(
        (mask[n_pos, i_pos] * N + n_pos).astype(np.int32))


_IDX2D = _build_indices()


@functools.cache
def _make_sc_gather():
    @functools.partial(
        pl.kernel,
        mesh=plsc.VectorSubcoreMesh(core_axis_name="c", subcore_axis_name="s"),
        out_type=jax.ShapeDtypeStruct((ROWS, D), jnp.float32),
        scratch_types=[
            pltpu.VMEM((PER_W,), jnp.int32),
            pltpu.VMEM((CHUNK, D), jnp.float32),
            pltpu.VMEM((CHUNK, D), jnp.float32),
            pltpu.SemaphoreType.DMA,
            pltpu.SemaphoreType.DMA,
            pltpu.SemaphoreType.DMA,
            pltpu.SemaphoreType.DMA,
        ],
    )
    def _sc_gather(x_hbm, idx_hbm, out_hbm, idx_v, buf0, buf1, g0, g1, s0, s1):
        wid = lax.axis_index("s") * 2 + lax.axis_index("c")
        base = (STRIDE * wid) // 8 * 8
        pltpu.sync_copy(idx_hbm.at[wid], idx_v)

        bufs = (buf0, buf1)
        gsems = (g0, g1)
        ssems = (s0, s1)
        gather_h = [None, None]
        scatter_h = [None, None]

        def start_gather(c):
            b = c & 1
            sz = CHUNK_SIZES[c]
            idx_slice = idx_v.at[pl.ds(CHUNK_OFFS[c], sz)]
            gather_h[b] = pltpu.async_copy(
                x_hbm.at[idx_slice], bufs[b].at[pl.ds(0, sz)], gsems[b])

        start_gather(0)
        for c in range(9):
            b = c & 1
            gather_h[b].wait()
            if c + 1 < 9:
                if scatter_h[1 - b] is not None:
                    scatter_h[1 - b].wait()
                start_gather(c + 1)
            sz = CHUNK_SIZES[c]
            scatter_h[b] = pltpu.async_copy(
                bufs[b].at[pl.ds(0, sz)],
                out_hbm.at[pl.ds(base + CHUNK_OFFS[c], sz)],
                ssems[b])
        scatter_h[0].wait()
        scatter_h[1].wait()

    return _sc_gather


def kernel(x):
    x_t = jnp.transpose(x, (1, 0, 2)).reshape(L * N, D)
    idx2d = jnp.asarray(_IDX2D)
    out = _make_sc_gather()(x_t, idx2d)
    return jnp.transpose(out.reshape(OUT_L, N, D), (1, 0, 2))


# speedup vs baseline: 5.8622x; 1.0194x over previous
import functools

import numpy as np
import jax
import jax.numpy as jnp
from jax import lax
from jax.experimental import pallas as pl
from jax.experimental.pallas import tpu as pltpu
from jax.experimental.pallas import tpu_sc as plsc

N, L, D = 64, 577, 768
KEEP = 288
OUT_L = KEEP + 1
ROWS = N * OUT_L
NW = 32
PER_W = 584
STRIDE = 578
CHUNK = 40
NFULL = 14
CHUNK_SIZES = [CHUNK] * NFULL + [PER_W - NFULL * CHUNK]
CHUNK_OFFS = [i * CHUNK for i in range(NFULL + 1)]
NCHUNK = NFULL + 1
NBUF = 4


def _rotl32(x, d):
    return ((x << np.uint32(d)) | (x >> np.uint32(32 - d))).astype(np.uint32)


def _threefry2x32(k0, k1, x0, x1):
    rot = ((13, 15, 26, 6), (17, 29, 16, 24))
    ks = (np.uint32(k0), np.uint32(k1),
          np.uint32(np.uint32(k0) ^ np.uint32(k1) ^ np.uint32(0x1BD11BDA)))
    x0 = (x0 + ks[0]).astype(np.uint32)
    x1 = (x1 + ks[1]).astype(np.uint32)
    for i in range(5):
        for r in rot[i % 2]:
            x0 = (x0 + x1).astype(np.uint32)
            x1 = _rotl32(x1, r)
            x1 = (x0 ^ x1).astype(np.uint32)
        x0 = (x0 + ks[(i + 1) % 3]).astype(np.uint32)
        x1 = (x1 + ks[(i + 2) % 3] + np.uint32(i + 1)).astype(np.uint32)
    return x0, x1


def _uniform_key1(shape):
    n = int(np.prod(shape))
    hi = np.zeros(n, dtype=np.uint32)
    lo = np.arange(n, dtype=np.uint32)
    o0, o1 = _threefry2x32(0, 1, hi, lo)
    bits = (o0 ^ o1).astype(np.uint32)
    fb = (bits >> np.uint32(9)) | np.uint32(0x3F800000)
    return (fb.view(np.float32) - np.float32(1.0)).reshape(shape)


def _build_indices():
    scores = _uniform_key1((N, L - 1))
    pm = np.argsort(scores, axis=1, kind="stable")[:, :KEEP] + 1
    pm.sort(axis=1)
    mask = np.concatenate([np.zeros((N, 1), pm.dtype), pm], axis=1)
    bases = (STRIDE * np.arange(NW)) // 8 * 8
    j = bases[:, None] + np.arange(PER_W)[None, :]
    i_pos = j // N
    n_pos = j % N
    return np.ascontiguousarray(
        (mask[n_pos, i_pos] * N + n_pos).astype(np.int32))


_IDX2D = _build_indices()


@functools.cache
def _make_sc_gather():
    @functools.partial(
        pl.kernel,
        mesh=plsc.VectorSubcoreMesh(core_axis_name="c", subcore_axis_name="s"),
        out_type=jax.ShapeDtypeStruct((ROWS, D), jnp.float32),
        scratch_types=(
            [pltpu.VMEM((PER_W,), jnp.int32)]
            + [pltpu.VMEM((CHUNK, D), jnp.float32) for _ in range(NBUF)]
            + [pltpu.SemaphoreType.DMA for _ in range(2 * NBUF)]
        ),
    )
    def _sc_gather(x_hbm, idx_hbm, out_hbm, idx_v, *bufs_sems):
        bufs = bufs_sems[:NBUF]
        gsems = bufs_sems[NBUF:2 * NBUF]
        ssems = bufs_sems[2 * NBUF:]
        wid = lax.axis_index("s") * 2 + lax.axis_index("c")
        base = (STRIDE * wid) // 8 * 8
        pltpu.sync_copy(idx_hbm.at[wid], idx_v)

        gather_h = [None] * NBUF
        scatter_h = [None] * NBUF

        def start_gather(c):
            b = c % NBUF
            sz = CHUNK_SIZES[c]
            idx_slice = idx_v.at[pl.ds(CHUNK_OFFS[c], sz)]
            gather_h[b] = pltpu.async_copy(
                x_hbm.at[idx_slice], bufs[b].at[pl.ds(0, sz)], gsems[b])

        for c in range(min(NBUF - 1, NCHUNK)):
            start_gather(c)
        for c in range(NCHUNK):
            b = c % NBUF
            gather_h[b].wait()
            sz = CHUNK_SIZES[c]
            scatter_h[b] = pltpu.async_copy(
                bufs[b].at[pl.ds(0, sz)],
                out_hbm.at[pl.ds(base + CHUNK_OFFS[c], sz)],
                ssems[b])
            nxt = c + NBUF - 1
            if nxt < NCHUNK:
                nb = nxt % NBUF
                if scatter_h[nb] is not None:
                    scatter_h[nb].wait()
                start_gather(nxt)
        for h in scatter_h:
            if h is not None:
                h.wait()

    return _sc_gather


def kernel(x):
    x_t = jnp.transpose(x, (1, 0, 2)).reshape(L * N, D)
    idx2d = jnp.asarray(_IDX2D)
    out = _make_sc_gather()(x_t, idx2d)
    return jnp.transpose(out.reshape(OUT_L, N, D), (1, 0, 2))
